# baseline (speedup 1.0000x reference)
import jax
import jax.numpy as jnp
from jax import lax
from jax.experimental import pallas as pl
from jax.experimental.pallas import tpu as pltpu
from jax.experimental.pallas import tpu_sc as plsc

_B = 4
_S = 8192
_D = 2048
_NE = 64
_NR = 4
_TOPK = (8, 4, 4, 4)
_BST = 2048
_NS = (_B * _S) // _BST
_SPB = _S // _BST
_NROWS = _B * _NR
_KMAX = 8


def _router_kernel(x_ref, imp_ref, w_ref, dense_ref):
    step = pl.program_id(0)
    w = w_ref[...]
    m_rows = _BST

    x2 = x_ref[...]
    logits = lax.dot_general(
        x2, w, (((1,), (0,)), ((), ())),
        preferred_element_type=jnp.float32)
    e_all = jnp.exp(logits)
    nc = _NR * _NE
    gi = lax.broadcasted_iota(jnp.int32, (nc, nc), 0) // _NE
    gj = lax.broadcasted_iota(jnp.int32, (nc, nc), 1) // _NE
    gblock = (gi == gj).astype(jnp.float32)
    denom = lax.dot_general(
        e_all, gblock, (((1,), (0,)), ((), ())),
        precision=lax.Precision.HIGHEST,
        preferred_element_type=jnp.float32)
    pall = e_all / denom

    impf = imp_ref[...]
    bcur = step // _SPB
    rowb = lax.broadcasted_iota(jnp.int32, (_B, m_rows), 0)
    imp4 = jnp.where(rowb == bcur, jnp.broadcast_to(impf, (_B, m_rows)), 0.0)
    full = lax.dot_general(
        imp4, pall, (((1,), (0,)), ((), ())),
        preferred_element_type=jnp.float32)

    @pl.when(step == 0)
    def _():
        dense_ref[...] = jnp.zeros_like(dense_ref)

    dense_ref[...] += full


def _sparsify_sc_kernel(dense_hbm, sparse_hbm, idx_hbm, vin, vsp, vidx):
    c = lax.axis_index("c")
    s = lax.axis_index("s")
    wid = s * 2 + c

    @pl.when(wid < _NROWS)
    def _():
        pltpu.sync_copy(dense_hbm.at[wid], vin)
        r = wid % _NR
        lane = lax.iota(jnp.int32, 16)

        svals, sidx = [], []
        for ci in range(4):
            key = vin[pl.ds(16 * ci, 16)]
            sk, sv = plsc.sort_key_val(key, lane + 16 * ci, descending=True)
            svals.append(sk)
            sidx.append(sv)

        def merge(ka, ia, kb, ib):
            rkb = lax.rev(kb, (0,))
            rib = lax.rev(ib, (0,))
            m = ka >= rkb
            hk = jnp.where(m, ka, rkb)
            hi = jnp.where(m, ia, rib)
            return plsc.sort_key_val(hk, hi, descending=True)

        k01, i01 = merge(svals[0], sidx[0], svals[1], sidx[1])
        k23, i23 = merge(svals[2], sidx[2], svals[3], sidx[3])
        kt, it = merge(k01, i01, k23, i23)

        take = lane < jnp.where(r == 0, _TOPK[0], _TOPK[1])
        tot = jnp.where(take, kt, 0.0)
        gdn = lax.GatherDimensionNumbers(
            offset_dims=(), collapsed_slice_dims=(0,), start_index_map=(0,))
        for sh in (1, 2, 4, 8):
            perm = (lane + sh) & 15
            tot = tot + lax.gather(
                tot, perm[:, None], gdn, (1,),
                mode=lax.GatherScatterMode.PROMISE_IN_BOUNDS)
        norm = kt * (1.0 / (tot + 1e-8))

        for ci in range(4):
            vsp[pl.ds(16 * ci, 16)] = jnp.zeros((16,), jnp.float32)
            vidx[pl.ds(16 * ci, 16)] = jnp.zeros((16,), jnp.int32)
        plsc.store_scatter(vsp, [it], norm, mask=take)
        vidx[pl.ds(0, 16)] = it
        pltpu.sync_copy(vsp, sparse_hbm.at[wid])
        pltpu.sync_copy(vidx, idx_hbm.at[wid])


def kernel(x, importance, W_compress, W_expand_Q, W_expand_K, W_expand_V):
    w_all = jnp.concatenate(
        [W_compress, W_expand_Q, W_expand_K, W_expand_V], axis=0).T

    xf = x.reshape(_B * _S, _D)
    impf = importance.reshape(1, _B * _S)

    dense_out = pl.pallas_call(
        _router_kernel,
        grid=(_NS,),
        in_specs=[
            pl.BlockSpec((_BST, _D), lambda s: (s, 0)),
            pl.BlockSpec((1, _BST), lambda s: (0, s)),
            pl.BlockSpec((_D, _NR * _NE), lambda s: (0, 0)),
        ],
        out_specs=pl.BlockSpec((_B, _NR * _NE), lambda s: (0, 0)),
        out_shape=jax.ShapeDtypeStruct((_B, _NR * _NE), jnp.float32),
        compiler_params=pltpu.CompilerParams(
            dimension_semantics=("arbitrary",)),
    )(xf, impf, w_all)

    dense16 = dense_out.reshape(_NROWS, _NE)

    sparse16, idx16 = pl.kernel(
        _sparsify_sc_kernel,
        out_type=[
            jax.ShapeDtypeStruct((_NROWS, _NE), jnp.float32),
            jax.ShapeDtypeStruct((_NROWS, _NE), jnp.int32),
        ],
        mesh=plsc.VectorSubcoreMesh(core_axis_name="c", subcore_axis_name="s"),
        compiler_params=pltpu.CompilerParams(needs_layout_passes=False),
        scratch_types=[
            pltpu.VMEM((_NE,), jnp.float32),
            pltpu.VMEM((_NE,), jnp.float32),
            pltpu.VMEM((_NE,), jnp.int32),
        ],
    )(dense16)

    sparse = sparse16.reshape(_B, _NR, _NE)
    idx = idx16.reshape(_B, _NR, _NE)
    dense = dense16.reshape(_B, _NR, _NE)

    return (
        sparse[:, 0, :],
        sparse[:, 1, :],
        sparse[:, 2, :],
        sparse[:, 3, :],
        dense[:, 0, :],
        dense[:, 1, :],
        dense[:, 2, :],
        dense[:, 3, :],
        idx[:, 0, :_TOPK[0]],
        idx[:, 1, :_TOPK[1]],
        idx[:, 2, :_TOPK[2]],
        idx[:, 3, :_TOPK[3]],
    )


# speedup vs baseline: 1.3000x; 1.3000x over previous
import jax
import jax.numpy as jnp
from jax import lax
from jax.experimental import pallas as pl
from jax.experimental.pallas import tpu as pltpu
from jax.experimental.pallas import tpu_sc as plsc

_B = 4
_S = 8192
_D = 2048
_NE = 64
_NR = 4
_TOPK = (8, 4, 4, 4)
_BST = 2048
_NS = (_B * _S) // _BST
_SPB = _S // _BST
_NROWS = _B * _NR
_KMAX = 8


def _router_kernel(x_ref, imp_ref, w_ref, dense_ref):
    step = pl.program_id(0)
    w = w_ref[...]
    m_rows = _BST

    x2 = x_ref[...]
    logits = lax.dot_general(
        x2, w, (((1,), (0,)), ((), ())),
        preferred_element_type=jnp.float32)
    e_all = jnp.exp(logits)
    nc = _NR * _NE
    gi = lax.broadcasted_iota(jnp.int32, (nc, nc), 0) // _NE
    gj = lax.broadcasted_iota(jnp.int32, (nc, nc), 1) // _NE
    gblock = (gi == gj).astype(jnp.bfloat16)
    e_hi = e_all.astype(jnp.bfloat16)
    e_lo = (e_all - e_hi.astype(jnp.float32)).astype(jnp.bfloat16)
    denom = lax.dot_general(
        e_hi, gblock, (((1,), (0,)), ((), ())),
        preferred_element_type=jnp.float32) + lax.dot_general(
        e_lo, gblock, (((1,), (0,)), ((), ())),
        preferred_element_type=jnp.float32)
    pall = e_all / denom

    impf = imp_ref[...]
    bcur = step // _SPB
    rowb = lax.broadcasted_iota(jnp.int32, (_B, m_rows), 0)
    imp4 = jnp.where(rowb == bcur, jnp.broadcast_to(impf, (_B, m_rows)), 0.0)
    full = lax.dot_general(
        imp4, pall, (((1,), (0,)), ((), ())),
        preferred_element_type=jnp.float32)

    @pl.when(step == 0)
    def _():
        dense_ref[...] = jnp.zeros_like(dense_ref)

    dense_ref[...] += full


def _sparsify_sc_kernel(dense_hbm, sparse_hbm, idx_hbm, vin, vsp, vidx):
    c = lax.axis_index("c")
    s = lax.axis_index("s")
    wid = s * 2 + c

    @pl.when(wid < _NROWS)
    def _():
        pltpu.sync_copy(dense_hbm.at[wid], vin)
        r = wid % _NR
        lane = lax.iota(jnp.int32, 16)

        svals, sidx = [], []
        for ci in range(4):
            key = vin[pl.ds(16 * ci, 16)]
            sk, sv = plsc.sort_key_val(key, lane + 16 * ci, descending=True)
            svals.append(sk)
            sidx.append(sv)

        def merge(ka, ia, kb, ib):
            rkb = lax.rev(kb, (0,))
            rib = lax.rev(ib, (0,))
            m = ka >= rkb
            hk = jnp.where(m, ka, rkb)
            hi = jnp.where(m, ia, rib)
            return plsc.sort_key_val(hk, hi, descending=True)

        k01, i01 = merge(svals[0], sidx[0], svals[1], sidx[1])
        k23, i23 = merge(svals[2], sidx[2], svals[3], sidx[3])
        kt, it = merge(k01, i01, k23, i23)

        take = lane < jnp.where(r == 0, _TOPK[0], _TOPK[1])
        tot = jnp.where(take, kt, 0.0)
        gdn = lax.GatherDimensionNumbers(
            offset_dims=(), collapsed_slice_dims=(0,), start_index_map=(0,))
        for sh in (1, 2, 4, 8):
            perm = (lane + sh) & 15
            tot = tot + lax.gather(
                tot, perm[:, None], gdn, (1,),
                mode=lax.GatherScatterMode.PROMISE_IN_BOUNDS)
        norm = kt * (1.0 / (tot + 1e-8))

        for ci in range(4):
            vsp[pl.ds(16 * ci, 16)] = jnp.zeros((16,), jnp.float32)
            vidx[pl.ds(16 * ci, 16)] = jnp.zeros((16,), jnp.int32)
        plsc.store_scatter(vsp, [it], norm, mask=take)
        vidx[pl.ds(0, 16)] = it
        pltpu.sync_copy(vsp, sparse_hbm.at[wid])
        pltpu.sync_copy(vidx, idx_hbm.at[wid])


def kernel(x, importance, W_compress, W_expand_Q, W_expand_K, W_expand_V):
    w_all = jnp.concatenate(
        [W_compress, W_expand_Q, W_expand_K, W_expand_V], axis=0).T

    xf = x.reshape(_B * _S, _D)
    impf = importance.reshape(1, _B * _S)

    dense_out = pl.pallas_call(
        _router_kernel,
        grid=(_NS,),
        in_specs=[
            pl.BlockSpec((_BST, _D), lambda s: (s, 0)),
            pl.BlockSpec((1, _BST), lambda s: (0, s)),
            pl.BlockSpec((_D, _NR * _NE), lambda s: (0, 0)),
        ],
        out_specs=pl.BlockSpec((_B, _NR * _NE), lambda s: (0, 0)),
        out_shape=jax.ShapeDtypeStruct((_B, _NR * _NE), jnp.float32),
        compiler_params=pltpu.CompilerParams(
            dimension_semantics=("arbitrary",)),
    )(xf, impf, w_all)

    dense16 = dense_out.reshape(_NROWS, _NE)

    sparse16, idx16 = pl.kernel(
        _sparsify_sc_kernel,
        out_type=[
            jax.ShapeDtypeStruct((_NROWS, _NE), jnp.float32),
            jax.ShapeDtypeStruct((_NROWS, _NE), jnp.int32),
        ],
        mesh=plsc.VectorSubcoreMesh(core_axis_name="c", subcore_axis_name="s"),
        compiler_params=pltpu.CompilerParams(needs_layout_passes=False),
        scratch_types=[
            pltpu.VMEM((_NE,), jnp.float32),
            pltpu.VMEM((_NE,), jnp.float32),
            pltpu.VMEM((_NE,), jnp.int32),
        ],
    )(dense16)

    sparse = sparse16.reshape(_B, _NR, _NE)
    idx = idx16.reshape(_B, _NR, _NE)
    dense = dense16.reshape(_B, _NR, _NE)

    return (
        sparse[:, 0, :],
        sparse[:, 1, :],
        sparse[:, 2, :],
        sparse[:, 3, :],
        dense[:, 0, :],
        dense[:, 1, :],
        dense[:, 2, :],
        dense[:, 3, :],
        idx[:, 0, :_TOPK[0]],
        idx[:, 1, :_TOPK[1]],
        idx[:, 2, :_TOPK[2]],
        idx[:, 3, :_TOPK[3]],
    )
